# baseline (speedup 1.0000x reference)
import functools

import jax
import jax.numpy as jnp
from jax import lax
from jax.experimental import pallas as pl
from jax.experimental.pallas import tpu as pltpu
from jax.experimental.pallas import tpu_sc as plsc

CH = (104, 96)
VB = 2048


def _transpose_body(in_ref, out_ref):
    d = in_ref.shape[0]
    eye = (
        lax.broadcasted_iota(jnp.int32, (d, d), 0)
        == lax.broadcasted_iota(jnp.int32, (d, d), 1)
    ).astype(jnp.float32)
    out_ref[:, :d] = lax.dot_general(
        in_ref[...], eye, (((0,), (0,)), ((), ())),
        preferred_element_type=jnp.float32,
    )


@functools.lru_cache(maxsize=None)
def _make_padtr(V, D):
    return pl.pallas_call(
        _transpose_body,
        grid=(pl.cdiv(V, VB),),
        in_specs=[pl.BlockSpec((D, VB), lambda g: (0, g))],
        out_specs=pl.BlockSpec((VB, 2 * D), lambda g: (g, 0)),
        out_shape=jax.ShapeDtypeStruct((V, 2 * D), jnp.float32),
    )


@functools.lru_cache(maxsize=None)
def _make_pool(B, H, V, D):
    NC, NS, L = 2, 16, 16
    NW = NC * NS
    assert B % NW == 0
    bpw = B // NW
    assert H == CH[0] + CH[1]
    assert D % L == 0
    nv = D // L
    D2 = 2 * D
    cmax = CH[0]

    mesh = plsc.VectorSubcoreMesh(core_axis_name="c", subcore_axis_name="s")

    @functools.partial(
        pl.kernel,
        mesh=mesh,
        out_type=jax.ShapeDtypeStruct((B, D), jnp.float32),
        scratch_types=[
            pltpu.VMEM((2 * bpw, cmax), jnp.int32),
            pltpu.VMEM((2, cmax, D2), jnp.float32),
            pltpu.VMEM((bpw, D), jnp.float32),
            pltpu.SemaphoreType.DMA((2,)),
        ],
    )
    def pool(idx_hbm, table_hbm, out_hbm, idx_v, rows_v, out_v, sems):
        wid = lax.axis_index("s") * NC + lax.axis_index("c")
        base = wid * bpw
        pltpu.sync_copy(idx_hbm.at[pl.ds(2 * base, 2 * bpw)], idx_v)

        def issue(item, h):
            pltpu.async_copy(
                table_hbm.at[idx_v.at[2 * item + h, pl.ds(0, CH[h])]],
                rows_v.at[h, pl.ds(0, CH[h])],
                sems.at[h],
            )

        def drain(h):
            pltpu.make_async_copy(
                table_hbm.at[pl.ds(0, CH[h])],
                rows_v.at[h, pl.ds(0, CH[h])],
                sems.at[h],
            ).wait()

        for h in range(2):
            issue(0, h)

        inv = jnp.float32(1.0 / H)
        zero = jnp.zeros((L,), jnp.float32)

        def item_step(i, carry):
            accs = (zero,) * nv
            for h in range(2):
                drain(h)

                def body(j, accs, h=h):
                    return tuple(
                        accs[k] + rows_v[h, j, pl.ds(k * L, L)]
                        for k in range(nv)
                    )

                accs = lax.fori_loop(0, CH[h], body, accs)

                @pl.when(i + 1 < bpw)
                def _(h=h):
                    issue(i + 1, h)

            for k in range(nv):
                out_v[i, pl.ds(k * L, L)] = accs[k] * inv
            return carry

        lax.fori_loop(0, bpw, item_step, 0)
        pltpu.sync_copy(out_v, out_hbm.at[pl.ds(base, bpw)])

    return pool


def _linear_body(p_ref, wt_ref, b_ref, o_ref):
    o_ref[...] = (
        jnp.dot(p_ref[...], wt_ref[...], preferred_element_type=jnp.float32)
        + b_ref[...]
    )


@functools.lru_cache(maxsize=None)
def _make_linear(B, D, O):
    return pl.pallas_call(
        _linear_body,
        out_shape=jax.ShapeDtypeStruct((B, O), jnp.float32),
    )


def kernel(x, table, W, b):
    B, H = x.shape
    V, D = table.shape
    O = W.shape[0]
    xi = x.astype(jnp.int32)
    x_c0 = xi[:, : CH[0]]
    x_c1 = jnp.pad(xi[:, CH[0] :], ((0, 0), (0, CH[0] - CH[1])))
    idx2 = jnp.stack([x_c0, x_c1], axis=1).reshape(2 * B, CH[0])
    table_pad = _make_padtr(V, D)(table.T)
    pooled = _make_pool(B, H, V, D)(idx2, table_pad)
    return _make_linear(B, D, O)(pooled, W.T, b[None, :])


# speedup vs baseline: 1.4222x; 1.4222x over previous
import functools

import jax
import jax.numpy as jnp
from jax import lax
from jax.experimental import pallas as pl
from jax.experimental.pallas import tpu as pltpu
from jax.experimental.pallas import tpu_sc as plsc

CH = (104, 96)
VB = 8192


def _transpose_body(in_ref, out_ref):
    d = in_ref.shape[0]
    out_ref[:, :d] = in_ref[...].T


@functools.lru_cache(maxsize=None)
def _make_padtr(V, D):
    return pl.pallas_call(
        _transpose_body,
        grid=(pl.cdiv(V, VB),),
        in_specs=[pl.BlockSpec((D, VB), lambda g: (0, g))],
        out_specs=pl.BlockSpec((VB, 2 * D), lambda g: (g, 0)),
        out_shape=jax.ShapeDtypeStruct((V, 2 * D), jnp.float32),
    )


@functools.lru_cache(maxsize=None)
def _make_pool(B, H, V, D):
    NC, NS, L = 2, 16, 16
    NW = NC * NS
    assert B % NW == 0
    bpw = B // NW
    assert H == CH[0] + CH[1]
    assert D % L == 0
    nv = D // L
    D2 = 2 * D
    cmax = CH[0]

    mesh = plsc.VectorSubcoreMesh(core_axis_name="c", subcore_axis_name="s")

    @functools.partial(
        pl.kernel,
        mesh=mesh,
        out_type=jax.ShapeDtypeStruct((B, D), jnp.float32),
        scratch_types=[
            pltpu.VMEM((2 * bpw, cmax), jnp.int32),
            pltpu.VMEM((2, cmax, D2), jnp.float32),
            pltpu.VMEM((bpw, D), jnp.float32),
            pltpu.SemaphoreType.DMA((2,)),
        ],
    )
    def pool(idx_hbm, table_hbm, out_hbm, idx_v, rows_v, out_v, sems):
        wid = lax.axis_index("s") * NC + lax.axis_index("c")
        base = wid * bpw
        pltpu.sync_copy(idx_hbm.at[pl.ds(2 * base, 2 * bpw)], idx_v)

        def issue(item, h):
            pltpu.async_copy(
                table_hbm.at[idx_v.at[2 * item + h, pl.ds(0, CH[h])]],
                rows_v.at[h, pl.ds(0, CH[h])],
                sems.at[h],
            )

        def drain(h):
            pltpu.make_async_copy(
                table_hbm.at[pl.ds(0, CH[h])],
                rows_v.at[h, pl.ds(0, CH[h])],
                sems.at[h],
            ).wait()

        for h in range(2):
            issue(0, h)

        inv = jnp.float32(1.0 / H)
        zero = jnp.zeros((L,), jnp.float32)

        def item_step(i, carry):
            accs = (zero,) * nv
            for h in range(2):
                drain(h)

                def body(j, accs, h=h):
                    return tuple(
                        accs[k] + rows_v[h, j, pl.ds(k * L, L)]
                        for k in range(nv)
                    )

                accs = lax.fori_loop(0, CH[h], body, accs)

                @pl.when(i + 1 < bpw)
                def _(h=h):
                    issue(i + 1, h)

            for k in range(nv):
                out_v[i, pl.ds(k * L, L)] = accs[k] * inv
            return carry

        lax.fori_loop(0, bpw, item_step, 0)
        pltpu.sync_copy(out_v, out_hbm.at[pl.ds(base, bpw)])

    return pool


def _linear_body(p_ref, wt_ref, b_ref, o_ref):
    o_ref[...] = (
        jnp.dot(p_ref[...], wt_ref[...], preferred_element_type=jnp.float32)
        + b_ref[...]
    )


@functools.lru_cache(maxsize=None)
def _make_linear(B, D, O):
    return pl.pallas_call(
        _linear_body,
        out_shape=jax.ShapeDtypeStruct((B, O), jnp.float32),
    )


def kernel(x, table, W, b):
    B, H = x.shape
    V, D = table.shape
    O = W.shape[0]
    xi = x.astype(jnp.int32)
    x_c0 = xi[:, : CH[0]]
    x_c1 = jnp.pad(xi[:, CH[0] :], ((0, 0), (0, CH[0] - CH[1])))
    idx2 = jnp.stack([x_c0, x_c1], axis=1).reshape(2 * B, CH[0])
    table_pad = _make_padtr(V, D)(table.T)
    pooled = _make_pool(B, H, V, D)(idx2, table_pad)
    return _make_linear(B, D, O)(pooled, W.T, b[None, :])


# speedup vs baseline: 1.4759x; 1.0378x over previous
import functools

import jax
import jax.numpy as jnp
from jax import lax
from jax.experimental import pallas as pl
from jax.experimental.pallas import tpu as pltpu
from jax.experimental.pallas import tpu_sc as plsc

CH = (104, 96)
VB = 16384


def _transpose_body(in_ref, out_ref):
    d = in_ref.shape[0]
    out_ref[:, :d] = in_ref[...].T


@functools.lru_cache(maxsize=None)
def _make_padtr(V, D):
    return pl.pallas_call(
        _transpose_body,
        grid=(pl.cdiv(V, VB),),
        in_specs=[pl.BlockSpec((D, VB), lambda g: (0, g))],
        out_specs=pl.BlockSpec((VB, 2 * D), lambda g: (g, 0)),
        out_shape=jax.ShapeDtypeStruct((V, 2 * D), jnp.float32),
    )


@functools.lru_cache(maxsize=None)
def _make_pool(B, H, V, D):
    NC, NS, L = 2, 16, 16
    NW = NC * NS
    assert B % NW == 0
    bpw = B // NW
    assert H == CH[0] + CH[1]
    assert D % L == 0
    nv = D // L
    D2 = 2 * D
    cmax = CH[0]

    mesh = plsc.VectorSubcoreMesh(core_axis_name="c", subcore_axis_name="s")

    @functools.partial(
        pl.kernel,
        mesh=mesh,
        out_type=jax.ShapeDtypeStruct((B, D), jnp.float32),
        scratch_types=[
            pltpu.VMEM((2 * bpw, cmax), jnp.int32),
            pltpu.VMEM((2, cmax, D2), jnp.float32),
            pltpu.VMEM((bpw, D), jnp.float32),
            pltpu.SemaphoreType.DMA((2,)),
        ],
    )
    def pool(idx_hbm, table_hbm, out_hbm, idx_v, rows_v, out_v, sems):
        wid = lax.axis_index("s") * NC + lax.axis_index("c")
        base = wid * bpw
        pltpu.sync_copy(idx_hbm.at[pl.ds(2 * base, 2 * bpw)], idx_v)

        def issue(item, h):
            pltpu.async_copy(
                table_hbm.at[idx_v.at[2 * item + h, pl.ds(0, CH[h])]],
                rows_v.at[h, pl.ds(0, CH[h])],
                sems.at[h],
            )

        def drain(h):
            pltpu.make_async_copy(
                table_hbm.at[pl.ds(0, CH[h])],
                rows_v.at[h, pl.ds(0, CH[h])],
                sems.at[h],
            ).wait()

        for h in range(2):
            issue(0, h)

        inv = jnp.float32(1.0 / H)
        zero = jnp.zeros((L,), jnp.float32)

        def item_step(i, carry):
            accs = (zero,) * nv
            for h in range(2):
                drain(h)

                def body(j, accs, h=h):
                    return tuple(
                        accs[k] + rows_v[h, j, pl.ds(k * L, L)]
                        for k in range(nv)
                    )

                accs = lax.fori_loop(0, CH[h], body, accs)

                @pl.when(i + 1 < bpw)
                def _(h=h):
                    issue(i + 1, h)

            for k in range(nv):
                out_v[i, pl.ds(k * L, L)] = accs[k] * inv
            return carry

        lax.fori_loop(0, bpw, item_step, 0)
        pltpu.sync_copy(out_v, out_hbm.at[pl.ds(base, bpw)])

    return pool


def _linear_body(p_ref, wt_ref, b_ref, o_ref):
    o_ref[...] = (
        jnp.dot(p_ref[...], wt_ref[...], preferred_element_type=jnp.float32)
        + b_ref[...]
    )


@functools.lru_cache(maxsize=None)
def _make_linear(B, D, O):
    return pl.pallas_call(
        _linear_body,
        out_shape=jax.ShapeDtypeStruct((B, O), jnp.float32),
    )


def kernel(x, table, W, b):
    B, H = x.shape
    V, D = table.shape
    O = W.shape[0]
    xi = x.astype(jnp.int32)
    x_c0 = xi[:, : CH[0]]
    x_c1 = jnp.pad(xi[:, CH[0] :], ((0, 0), (0, CH[0] - CH[1])))
    idx2 = jnp.stack([x_c0, x_c1], axis=1).reshape(2 * B, CH[0])
    table_pad = _make_padtr(V, D)(table.T)
    pooled = _make_pool(B, H, V, D)(idx2, table_pad)
    return _make_linear(B, D, O)(pooled, W.T, b[None, :])


# speedup vs baseline: 1.4980x; 1.0150x over previous
import functools

import jax
import jax.numpy as jnp
from jax import lax
from jax.experimental import pallas as pl
from jax.experimental.pallas import tpu as pltpu
from jax.experimental.pallas import tpu_sc as plsc

CH = (104, 96)
VB = 32768


def _transpose_body(in_ref, out_ref):
    d = in_ref.shape[0]
    out_ref[:, :d] = in_ref[...].T


@functools.lru_cache(maxsize=None)
def _make_padtr(V, D):
    return pl.pallas_call(
        _transpose_body,
        grid=(pl.cdiv(V, VB),),
        in_specs=[pl.BlockSpec((D, VB), lambda g: (0, g))],
        out_specs=pl.BlockSpec((VB, 2 * D), lambda g: (g, 0)),
        out_shape=jax.ShapeDtypeStruct((V, 2 * D), jnp.float32),
    )


@functools.lru_cache(maxsize=None)
def _make_pool(B, H, V, D):
    NC, NS, L = 2, 16, 16
    NW = NC * NS
    assert B % NW == 0
    bpw = B // NW
    assert H == CH[0] + CH[1]
    assert D % L == 0
    nv = D // L
    D2 = 2 * D
    cmax = CH[0]

    mesh = plsc.VectorSubcoreMesh(core_axis_name="c", subcore_axis_name="s")

    @functools.partial(
        pl.kernel,
        mesh=mesh,
        out_type=jax.ShapeDtypeStruct((B, D), jnp.float32),
        scratch_types=[
            pltpu.VMEM((2 * bpw, cmax), jnp.int32),
            pltpu.VMEM((2, cmax, D2), jnp.float32),
            pltpu.VMEM((bpw, D), jnp.float32),
            pltpu.SemaphoreType.DMA((2,)),
        ],
    )
    def pool(idx_hbm, table_hbm, out_hbm, idx_v, rows_v, out_v, sems):
        wid = lax.axis_index("s") * NC + lax.axis_index("c")
        base = wid * bpw
        pltpu.sync_copy(idx_hbm.at[pl.ds(2 * base, 2 * bpw)], idx_v)

        def issue(item, h):
            pltpu.async_copy(
                table_hbm.at[idx_v.at[2 * item + h, pl.ds(0, CH[h])]],
                rows_v.at[h, pl.ds(0, CH[h])],
                sems.at[h],
            )

        def drain(h):
            pltpu.make_async_copy(
                table_hbm.at[pl.ds(0, CH[h])],
                rows_v.at[h, pl.ds(0, CH[h])],
                sems.at[h],
            ).wait()

        for h in range(2):
            issue(0, h)

        inv = jnp.float32(1.0 / H)
        zero = jnp.zeros((L,), jnp.float32)

        def item_step(i, carry):
            accs = (zero,) * nv
            for h in range(2):
                drain(h)

                def body(j, accs, h=h):
                    return tuple(
                        accs[k] + rows_v[h, j, pl.ds(k * L, L)]
                        for k in range(nv)
                    )

                accs = lax.fori_loop(0, CH[h], body, accs)

                @pl.when(i + 1 < bpw)
                def _(h=h):
                    issue(i + 1, h)

            for k in range(nv):
                out_v[i, pl.ds(k * L, L)] = accs[k] * inv
            return carry

        lax.fori_loop(0, bpw, item_step, 0)
        pltpu.sync_copy(out_v, out_hbm.at[pl.ds(base, bpw)])

    return pool


def _linear_body(p_ref, wt_ref, b_ref, o_ref):
    o_ref[...] = (
        jnp.dot(p_ref[...], wt_ref[...], preferred_element_type=jnp.float32)
        + b_ref[...]
    )


@functools.lru_cache(maxsize=None)
def _make_linear(B, D, O):
    return pl.pallas_call(
        _linear_body,
        out_shape=jax.ShapeDtypeStruct((B, O), jnp.float32),
    )


def kernel(x, table, W, b):
    B, H = x.shape
    V, D = table.shape
    O = W.shape[0]
    xi = x.astype(jnp.int32)
    x_c0 = xi[:, : CH[0]]
    x_c1 = jnp.pad(xi[:, CH[0] :], ((0, 0), (0, CH[0] - CH[1])))
    idx2 = jnp.stack([x_c0, x_c1], axis=1).reshape(2 * B, CH[0])
    table_pad = _make_padtr(V, D)(table.T)
    pooled = _make_pool(B, H, V, D)(idx2, table_pad)
    return _make_linear(B, D, O)(pooled, W.T, b[None, :])


# speedup vs baseline: 1.8066x; 1.2059x over previous
import functools

import jax
import jax.numpy as jnp
from jax import lax
from jax.experimental import pallas as pl
from jax.experimental.pallas import tpu as pltpu
from jax.experimental.pallas import tpu_sc as plsc

VB = 32768
NBUF = 2


def _transpose_body(in_ref, out_ref):
    d = in_ref.shape[0]
    out_ref[:, :d] = in_ref[...].T


@functools.lru_cache(maxsize=None)
def _make_padtr(V, D):
    return pl.pallas_call(
        _transpose_body,
        grid=(pl.cdiv(V, VB),),
        in_specs=[pl.BlockSpec((D, VB), lambda g: (0, g))],
        out_specs=pl.BlockSpec((VB, 2 * D), lambda g: (g, 0)),
        out_shape=jax.ShapeDtypeStruct((V, 2 * D), jnp.float32),
    )


@functools.lru_cache(maxsize=None)
def _make_pool(B, H, V, D):
    NC, NS, L = 2, 16, 16
    NW = NC * NS
    assert B % NW == 0
    bpw = B // NW
    assert H % 2 == 0 and H // 2 <= 128
    ch = H // 2
    assert D % L == 0
    nv = D // L

    mesh = plsc.VectorSubcoreMesh(core_axis_name="c", subcore_axis_name="s")

    @functools.partial(
        pl.kernel,
        mesh=mesh,
        compiler_params=pltpu.CompilerParams(use_tc_tiling_on_sc=False),
        out_type=jax.ShapeDtypeStruct((B, D), jnp.float32),
        scratch_types=[
            pltpu.VMEM((bpw, 2, ch), jnp.int32),
            pltpu.VMEM((NBUF, H, D), jnp.float32),
            pltpu.VMEM((bpw, D), jnp.float32),
            pltpu.SemaphoreType.DMA((NBUF,)),
        ],
    )
    def pool(x_hbm, table_hbm, out_hbm, idx_v, rows_v, out_v, sems):
        wid = lax.axis_index("s") * NC + lax.axis_index("c")
        base = wid * bpw
        pltpu.sync_copy(x_hbm.at[pl.ds(base, bpw)], idx_v)

        def issue(item, p):
            for h in range(2):
                pltpu.async_copy(
                    table_hbm.at[idx_v.at[item, h]],
                    rows_v.at[p, pl.ds(h * ch, ch)],
                    sems.at[p],
                )

        def drain(p):
            pltpu.make_async_copy(
                table_hbm.at[pl.ds(0, H)], rows_v.at[p], sems.at[p]
            ).wait()

        for p in range(NBUF):
            issue(p, p)

        inv = jnp.float32(1.0 / H)
        zero = jnp.zeros((L,), jnp.float32)

        def outer(g, carry):
            for p in range(NBUF):
                i = g * NBUF + p
                drain(p)

                def body(j, accs):
                    return tuple(
                        accs[k] + rows_v[p, j, pl.ds(k * L, L)]
                        for k in range(nv)
                    )

                accs = lax.fori_loop(0, H, body, (zero,) * nv)

                nxt = i + NBUF

                @pl.when(nxt < bpw)
                def _():
                    issue(nxt, p)

                for k in range(nv):
                    out_v[i, pl.ds(k * L, L)] = accs[k] * inv
            return carry

        lax.fori_loop(0, bpw // NBUF, outer, 0)
        pltpu.sync_copy(out_v, out_hbm.at[pl.ds(base, bpw)])

    return pool


def _linear_body(p_ref, wt_ref, b_ref, o_ref):
    o_ref[...] = (
        jnp.dot(p_ref[...], wt_ref[...], preferred_element_type=jnp.float32)
        + b_ref[...]
    )


@functools.lru_cache(maxsize=None)
def _make_linear(B, D, O):
    return pl.pallas_call(
        _linear_body,
        out_shape=jax.ShapeDtypeStruct((B, O), jnp.float32),
    )


def kernel(x, table, W, b):
    B, H = x.shape
    V, D = table.shape
    O = W.shape[0]
    x3 = (x.astype(jnp.int32) << 1).reshape(B, 2, H // 2)
    table_lin = _make_padtr(V, D)(table.T).reshape(2 * V, D)
    pooled = _make_pool(B, H, V, D)(x3, table_lin)
    return _make_linear(B, D, O)(pooled, W.T, b[None, :])


# speedup vs baseline: 1.8638x; 1.0317x over previous
import functools

import jax
import jax.numpy as jnp
from jax import lax
from jax.experimental import pallas as pl
from jax.experimental.pallas import tpu as pltpu
from jax.experimental.pallas import tpu_sc as plsc

VB = 32768
NBUF = 2


def _transpose_body(in_ref, out_ref):
    d = in_ref.shape[0]
    out_ref[:, :d] = in_ref[...].T


@functools.lru_cache(maxsize=None)
def _make_padtr(V, D):
    return pl.pallas_call(
        _transpose_body,
        grid=(pl.cdiv(V, VB),),
        in_specs=[pl.BlockSpec((D, VB), lambda g: (0, g))],
        out_specs=pl.BlockSpec((VB, 2 * D), lambda g: (g, 0)),
        out_shape=jax.ShapeDtypeStruct((V, 2 * D), jnp.float32),
    )


@functools.lru_cache(maxsize=None)
def _make_pool(B, H, V, D):
    NC, NS, L = 2, 16, 16
    NW = NC * NS
    assert B % NW == 0
    bpw = B // NW
    assert H % 2 == 0 and H // 2 <= 128
    ch = H // 2
    assert D % L == 0
    nv = D // L

    mesh = plsc.VectorSubcoreMesh(core_axis_name="c", subcore_axis_name="s")

    @functools.partial(
        pl.kernel,
        mesh=mesh,
        compiler_params=pltpu.CompilerParams(use_tc_tiling_on_sc=False),
        out_type=jax.ShapeDtypeStruct((B, D), jnp.float32),
        scratch_types=[
            pltpu.VMEM((bpw, 2, ch), jnp.int32),
            pltpu.VMEM((NBUF, H, D), jnp.float32),
            pltpu.VMEM((bpw, D), jnp.float32),
            pltpu.SemaphoreType.DMA((NBUF,)),
        ],
    )
    def pool(x_hbm, table_hbm, out_hbm, idx_v, rows_v, out_v, sems):
        wid = lax.axis_index("s") * NC + lax.axis_index("c")
        base = wid * bpw
        pltpu.sync_copy(x_hbm.at[pl.ds(base, bpw)], idx_v)

        def issue(item, p):
            for h in range(2):
                pltpu.async_copy(
                    table_hbm.at[idx_v.at[item, h]],
                    rows_v.at[p, pl.ds(h * ch, ch)],
                    sems.at[p],
                )

        def drain(p):
            pltpu.make_async_copy(
                table_hbm.at[pl.ds(0, H)], rows_v.at[p], sems.at[p]
            ).wait()

        for p in range(NBUF):
            issue(p, p)

        inv = jnp.float32(1.0 / H)
        zero = jnp.zeros((L,), jnp.float32)

        def outer(g, carry):
            for p in range(NBUF):
                i = g * NBUF + p
                drain(p)

                def body(j, accs):
                    return tuple(
                        accs[k] + rows_v[p, j, pl.ds(k * L, L)]
                        for k in range(nv)
                    )

                accs = lax.fori_loop(0, H, body, (zero,) * nv, unroll=8)

                nxt = i + NBUF

                @pl.when(nxt < bpw)
                def _():
                    issue(nxt, p)

                for k in range(nv):
                    out_v[i, pl.ds(k * L, L)] = accs[k] * inv
            return carry

        lax.fori_loop(0, bpw // NBUF, outer, 0)
        pltpu.sync_copy(out_v, out_hbm.at[pl.ds(base, bpw)])

    return pool


def _linear_body(p_ref, wt_ref, b_ref, o_ref):
    o_ref[...] = (
        jnp.dot(p_ref[...], wt_ref[...], preferred_element_type=jnp.float32)
        + b_ref[...]
    )


@functools.lru_cache(maxsize=None)
def _make_linear(B, D, O):
    return pl.pallas_call(
        _linear_body,
        out_shape=jax.ShapeDtypeStruct((B, O), jnp.float32),
    )


def kernel(x, table, W, b):
    B, H = x.shape
    V, D = table.shape
    O = W.shape[0]
    x3 = (x.astype(jnp.int32) << 1).reshape(B, 2, H // 2)
    table_lin = _make_padtr(V, D)(table.T).reshape(2 * V, D)
    pooled = _make_pool(B, H, V, D)(x3, table_lin)
    return _make_linear(B, D, O)(pooled, W.T, b[None, :])


# speedup vs baseline: 2.0587x; 1.1046x over previous
import functools

import jax
import jax.numpy as jnp
from jax import lax
from jax.experimental import pallas as pl
from jax.experimental.pallas import tpu as pltpu
from jax.experimental.pallas import tpu_sc as plsc

VB = 32768
NBUF = 4


def _transpose_body(in_ref, out_ref):
    d = in_ref.shape[0]
    out_ref[:, :d] = in_ref[...].T


@functools.lru_cache(maxsize=None)
def _make_padtr(V, D):
    return pl.pallas_call(
        _transpose_body,
        grid=(pl.cdiv(V, VB),),
        in_specs=[pl.BlockSpec((D, VB), lambda g: (0, g))],
        out_specs=pl.BlockSpec((VB, 2 * D), lambda g: (g, 0)),
        out_shape=jax.ShapeDtypeStruct((V, 2 * D), jnp.float32),
    )


@functools.lru_cache(maxsize=None)
def _make_pool(B, H, V, D):
    NC, NS, L = 2, 16, 16
    NW = NC * NS
    assert B % NW == 0
    bpw = B // NW
    assert H % 2 == 0 and H // 2 <= 128
    ch = H // 2
    assert D % L == 0
    nv = D // L

    mesh = plsc.VectorSubcoreMesh(core_axis_name="c", subcore_axis_name="s")

    @functools.partial(
        pl.kernel,
        mesh=mesh,
        compiler_params=pltpu.CompilerParams(use_tc_tiling_on_sc=False),
        out_type=jax.ShapeDtypeStruct((B, D), jnp.float32),
        scratch_types=[
            pltpu.VMEM((bpw, 2, ch), jnp.int32),
            pltpu.VMEM((NBUF, H, D), jnp.float32),
            pltpu.VMEM((bpw, D), jnp.float32),
            pltpu.SemaphoreType.DMA((NBUF,)),
        ],
    )
    def pool(x_hbm, table_hbm, out_hbm, idx_v, rows_v, out_v, sems):
        wid = lax.axis_index("s") * NC + lax.axis_index("c")
        base = wid * bpw
        pltpu.sync_copy(x_hbm.at[pl.ds(base, bpw)], idx_v)

        def issue(item, p):
            for h in range(2):
                pltpu.async_copy(
                    table_hbm.at[idx_v.at[item, h]],
                    rows_v.at[p, pl.ds(h * ch, ch)],
                    sems.at[p],
                )

        def drain(p):
            pltpu.make_async_copy(
                table_hbm.at[pl.ds(0, H)], rows_v.at[p], sems.at[p]
            ).wait()

        for p in range(NBUF):
            issue(p, p)

        inv = jnp.float32(1.0 / H)
        zero = jnp.zeros((L,), jnp.float32)

        def outer(g, carry):
            for p in range(NBUF):
                i = g * NBUF + p
                drain(p)

                def body(j, accs):
                    return tuple(
                        accs[k] + rows_v[p, j, pl.ds(k * L, L)]
                        for k in range(nv)
                    )

                accs = lax.fori_loop(0, H, body, (zero,) * nv, unroll=8)

                nxt = i + NBUF

                @pl.when(nxt < bpw)
                def _():
                    issue(nxt, p)

                for k in range(nv):
                    out_v[i, pl.ds(k * L, L)] = accs[k] * inv
            return carry

        lax.fori_loop(0, bpw // NBUF, outer, 0)
        pltpu.sync_copy(out_v, out_hbm.at[pl.ds(base, bpw)])

    return pool


def _linear_body(p_ref, wt_ref, b_ref, o_ref):
    o_ref[...] = (
        jnp.dot(p_ref[...], wt_ref[...], preferred_element_type=jnp.float32)
        + b_ref[...]
    )


@functools.lru_cache(maxsize=None)
def _make_linear(B, D, O):
    return pl.pallas_call(
        _linear_body,
        out_shape=jax.ShapeDtypeStruct((B, O), jnp.float32),
    )


def kernel(x, table, W, b):
    B, H = x.shape
    V, D = table.shape
    O = W.shape[0]
    x3 = (x.astype(jnp.int32) << 1).reshape(B, 2, H // 2)
    table_lin = _make_padtr(V, D)(table.T).reshape(2 * V, D)
    pooled = _make_pool(B, H, V, D)(x3, table_lin)
    return _make_linear(B, D, O)(pooled, W.T, b[None, :])
